# Initial kernel scaffold; baseline (speedup 1.0000x reference)
import jax
import jax.numpy as jnp
from jax.experimental import pallas as pl


def kernel(positions, species, senders, receivers, shifts, cell, emb, W1, b1, W2, b2, w_out):
    raise NotImplementedError("write your pallas kernel here")


# speedup vs baseline: 38.6899x; 38.6899x over previous
import functools

import jax
import jax.numpy as jnp
from jax import lax
from jax.experimental import pallas as pl
from jax.experimental.pallas import tpu as pltpu
from jax.experimental.pallas import tpu_sc as plsc

NC = 2
NSC = 16
NW = NC * NSC
LB = 4096


def _largest_divisor_leq(n, cap):
    for k in range(cap - cap % 8, 0, -8):
        if n % k == 0:
            return k
    return 8


def _iota16():
    return lax.iota(jnp.int32, 16)


def _phase_a_body(MR, RPW, nchunks, NPn,
                  pxh, pyh, pzh, sph, snd2_h, rcv2_h,
                  sx_h, sy_h, sz_h, sp_h, rx_h, ry_h, rz_h,
                  px_s, py_s, pz_s, sp_s,
                  idx_s, idx_r,
                  sxb, syb, szb, spb, rxb, ryb, rzb, stg):
    cid = lax.axis_index("c")
    sid = lax.axis_index("s")
    wid = sid * NC + cid
    cke = MR * 128
    ST = NPn // NSC

    for hsrc, sdst in ((pxh, px_s), (pyh, py_s), (pzh, pz_s), (sph, sp_s)):
        pltpu.sync_copy(hsrc.at[pl.ds(sid * ST, ST)], stg)
        pltpu.sync_copy(stg, sdst.at[pl.ds(sid * ST, ST)])
    plsc.subcore_barrier()

    def chunk(ci, carry):
        rowbase = wid * RPW + ci * MR
        pltpu.sync_copy(snd2_h.at[pl.ds(rowbase, MR)], idx_s)
        pltpu.sync_copy(rcv2_h.at[pl.ds(rowbase, MR)], idx_r)

        def gath(j, c2):
            off = j * 128
            for tab, dst in ((px_s, sxb), (py_s, syb), (pz_s, szb),
                             (sp_s, spb)):
                pltpu.sync_copy(tab.at[idx_s.at[j]],
                                dst.at[pl.ds(off, 128)])
            for tab, dst in ((px_s, rxb), (py_s, ryb), (pz_s, rzb)):
                pltpu.sync_copy(tab.at[idx_r.at[j]],
                                dst.at[pl.ds(off, 128)])
            return c2

        lax.fori_loop(0, MR, gath, 0)

        ebase = wid * (RPW * 128) + ci * cke
        for buf, oh in ((sxb, sx_h), (syb, sy_h), (szb, sz_h), (spb, sp_h),
                        (rxb, rx_h), (ryb, ry_h), (rzb, rz_h)):
            pltpu.sync_copy(buf, oh.at[pl.ds(ebase, cke)])
        return carry

    lax.fori_loop(0, nchunks, chunk, 0)


def _make_phase_a(NP_edges, MR, RPW, NPn):
    nchunks = RPW // MR
    cke = MR * 128
    mesh = plsc.VectorSubcoreMesh(core_axis_name="c", subcore_axis_name="s",
                                  num_cores=NC, num_subcores=NSC)
    f32 = jnp.float32
    out = [jax.ShapeDtypeStruct((NP_edges,), f32) for _ in range(7)]
    scratch = (
        [pltpu.VMEM_SHARED((NPn,), f32) for _ in range(4)]
        + [pltpu.VMEM((MR, 128), jnp.int32) for _ in range(2)]
        + [pltpu.VMEM((cke,), f32) for _ in range(7)]
        + [pltpu.VMEM((NPn // NSC,), f32)]
    )
    return pl.kernel(
        functools.partial(_phase_a_body, MR, RPW, nchunks, NPn),
        out_type=tuple(out), mesh=mesh, scratch_types=scratch)


def _sigm(x):
    return 1.0 / (1.0 + jnp.exp(-x))


def _phase_b_body(sx, sy, sz, spf, rx, ry, rz, shx, shy, shz,
                  cellT, w1c, b1c, w2t, b2c, embT, woc,
                  fxo, fyo, fzo, eno):
    sh3 = jnp.concatenate([shx[0], shy[0], shz[0]], axis=0)
    sc3 = jnp.dot(cellT[...], sh3, preferred_element_type=jnp.float32)
    wx = rx[0] - sx[0] + sc3[0:1, :] + 1e-12
    wy = ry[0] - sy[0] + sc3[1:2, :] + 1e-12
    wz = rz[0] - sz[0] + sc3[2:3, :] + 1e-12
    d2 = wx * wx + wy * wy + wz * wz
    d = jnp.sqrt(d2)

    u = jnp.dot(w1c[...], d, preferred_element_type=jnp.float32) + b1c[...]
    su = _sigm(u)
    a = u * su
    v = jnp.dot(w2t[...], a, preferred_element_type=jnp.float32) + b2c[...]
    sv = _sigm(v)
    h2 = v * sv

    sp = lax.bitcast_convert_type(spf[0], jnp.int32)
    onehot = (lax.broadcasted_iota(jnp.int32, (128,) + sp.shape[1:], 0)
              == sp).astype(jnp.float32)
    te = jnp.dot(embT[...] * woc[...], onehot,
                 preferred_element_type=jnp.float32)

    g = jnp.sum(h2 * te).reshape(1, 1)

    dsu = su * (1.0 + u * (1.0 - su))
    da = dsu * w1c[...]
    dv = jnp.dot(w2t[...], da, preferred_element_type=jnp.float32)
    dh2 = sv * (1.0 + v * (1.0 - sv)) * dv
    c = jnp.sum(dh2 * te, axis=0, keepdims=True)

    q = c / d
    fxo[0] = q * wx
    fyo[0] = q * wy
    fzo[0] = q * wz

    @pl.when(pl.program_id(0) == 0)
    def _():
        eno[...] = jnp.zeros((1, 1), jnp.float32)

    eno[...] += g


def _make_phase_b(NP_edges):
    grid = (NP_edges // LB,)
    f32 = jnp.float32

    def row(i):
        return (i, 0, 0)

    def fixed(i):
        return (0, 0)

    edge_spec = pl.BlockSpec((1, 1, LB), row)
    in_specs = [edge_spec] * 10 + [
        pl.BlockSpec((3, 3), fixed),
        pl.BlockSpec((16, 1), fixed),
        pl.BlockSpec((16, 1), fixed),
        pl.BlockSpec((16, 16), fixed),
        pl.BlockSpec((16, 1), fixed),
        pl.BlockSpec((16, 128), fixed),
        pl.BlockSpec((16, 1), fixed),
    ]
    out_specs = [edge_spec, edge_spec, edge_spec,
                 pl.BlockSpec((1, 1), fixed)]
    out_shape = [jax.ShapeDtypeStruct((NP_edges // LB, 1, LB), f32)] * 3 + [
        jax.ShapeDtypeStruct((1, 1), f32)]
    return pl.pallas_call(_phase_b_body, grid=grid, in_specs=in_specs,
                          out_specs=out_specs, out_shape=out_shape)


def _phase_c_body(CR, RPW, NPn,
                  fx2, fy2, fz2, snd2, rcv2, zeros_h,
                  outa_h, outb_h,
                  f4, fxc, fyc, fzc, sc2, rc2,
                  isx, isy, isz, irx, iry, irz, nbx, nby, nbz, zwb):
    cid = lax.axis_index("c")
    sid = lax.axis_index("s")
    wid = sid * NC + cid
    SW = NPn * 4 // NSC
    ZW = NPn // 32

    pltpu.sync_copy(zeros_h, zwb)

    def zero(k, c2):
        pltpu.sync_copy(zwb, f4.at[pl.ds(sid * SW + k * ZW, ZW)])
        return c2

    lax.fori_loop(0, 8, zero, 0)
    plsc.subcore_barrier()

    def chunk(ch, c2):
        rb = wid * RPW + ch * CR
        pltpu.sync_copy(fx2.at[pl.ds(rb, CR)], fxc)
        pltpu.sync_copy(fy2.at[pl.ds(rb, CR)], fyc)
        pltpu.sync_copy(fz2.at[pl.ds(rb, CR)], fzc)
        pltpu.sync_copy(snd2.at[pl.ds(rb, CR)], sc2)
        pltpu.sync_copy(rcv2.at[pl.ds(rb, CR)], rc2)

        def rowfn(rr, c3):
            def grp(g, c4):
                off = g * 16
                s4 = sc2[rr, pl.ds(off, 16)] * 4
                r4 = rc2[rr, pl.ds(off, 16)] * 4
                isx[pl.ds(off, 16)] = s4
                isy[pl.ds(off, 16)] = s4 + 1
                isz[pl.ds(off, 16)] = s4 + 2
                irx[pl.ds(off, 16)] = r4
                iry[pl.ds(off, 16)] = r4 + 1
                irz[pl.ds(off, 16)] = r4 + 2
                nbx[pl.ds(off, 16)] = -fxc[rr, pl.ds(off, 16)]
                nby[pl.ds(off, 16)] = -fyc[rr, pl.ds(off, 16)]
                nbz[pl.ds(off, 16)] = -fzc[rr, pl.ds(off, 16)]
                return c4

            lax.fori_loop(0, 8, grp, 0)
            pltpu.sync_copy(fxc.at[rr], f4.at[isx], add=True)
            pltpu.sync_copy(fyc.at[rr], f4.at[isy], add=True)
            pltpu.sync_copy(fzc.at[rr], f4.at[isz], add=True)
            pltpu.sync_copy(nbx, f4.at[irx], add=True)
            pltpu.sync_copy(nby, f4.at[iry], add=True)
            pltpu.sync_copy(nbz, f4.at[irz], add=True)
            return c3

        lax.fori_loop(0, CR, rowfn, 0)
        return c2

    lax.fori_loop(0, RPW // CR, chunk, 0)
    plsc.subcore_barrier()

    @pl.when(cid == 0)
    def _():
        def writeback(k, c2):
            pltpu.sync_copy(f4.at[pl.ds(sid * SW + k * ZW, ZW)], zwb)
            pltpu.sync_copy(zwb, outa_h.at[pl.ds(sid * SW + k * ZW, ZW)])
            return c2

        lax.fori_loop(0, 8, writeback, 0)

    @pl.when(cid == 1)
    def _():
        def writeback(k, c2):
            pltpu.sync_copy(f4.at[pl.ds(sid * SW + k * ZW, ZW)], zwb)
            pltpu.sync_copy(zwb, outb_h.at[pl.ds(sid * SW + k * ZW, ZW)])
            return c2

        lax.fori_loop(0, 8, writeback, 0)


def _make_phase_c(NPn, CR, RPW):
    mesh = plsc.VectorSubcoreMesh(core_axis_name="c", subcore_axis_name="s",
                                  num_cores=NC, num_subcores=NSC)
    f32 = jnp.float32
    scratch = (
        [pltpu.VMEM_SHARED((NPn * 4,), f32)]
        + [pltpu.VMEM((CR, 128), f32) for _ in range(3)]
        + [pltpu.VMEM((CR, 128), jnp.int32) for _ in range(2)]
        + [pltpu.VMEM((128,), jnp.int32) for _ in range(6)]
        + [pltpu.VMEM((128,), f32) for _ in range(3)]
        + [pltpu.VMEM((NPn // 32,), f32)]
    )
    return pl.kernel(
        functools.partial(_phase_c_body, CR, RPW, NPn),
        out_type=(jax.ShapeDtypeStruct((NPn * 4,), f32),
                  jax.ShapeDtypeStruct((NPn * 4,), f32)),
        mesh=mesh, scratch_types=scratch)


def kernel(positions, species, senders, receivers, shifts, cell,
           emb, W1, b1, W2, b2, w_out):
    f32 = jnp.float32
    N = positions.shape[0]
    E = senders.shape[0]
    S = emb.shape[0]

    EPQ = NW * 128 * 8
    EP = ((E + EPQ - 1) // EPQ) * EPQ
    RPW = EP // 128 // NW
    NPn = ((N + 511) // 512) * 512
    MR = _largest_divisor_leq(RPW, 32)
    CR = _largest_divisor_leq(RPW, 32)

    pad_e = EP - E
    snd_p = jnp.concatenate([senders, jnp.zeros((pad_e,), jnp.int32)])
    rcv_p = jnp.concatenate([receivers, jnp.zeros((pad_e,), jnp.int32)])
    shifts_p = jnp.concatenate([shifts, jnp.zeros((pad_e, 3), f32)], axis=0)
    snd2 = snd_p.reshape(EP // 128, 128)
    rcv2 = rcv_p.reshape(EP // 128, 128)

    pad_n = NPn - N
    padn = lambda a: jnp.concatenate([a, jnp.zeros((pad_n,), f32)])
    pxh = padn(positions[:, 0])
    pyh = padn(positions[:, 1])
    pzh = padn(positions[:, 2])
    sph = padn(lax.bitcast_convert_type(species, f32))

    sx, sy, sz, spf, rx, ry, rz = _make_phase_a(EP, MR, RPW, NPn)(
        pxh, pyh, pzh, sph, snd2, rcv2)

    r2 = (EP // LB, 1, LB)
    cellT = cell[0].T.astype(f32)
    w1c = W1.reshape(16, 1).astype(f32)
    b1c = b1.reshape(16, 1).astype(f32)
    w2t = W2.T.astype(f32)
    b2c = b2.reshape(16, 1).astype(f32)
    embT = jnp.zeros((16, 128), f32).at[:, :S].set(emb.T)
    woc = w_out.reshape(16, 1).astype(f32)

    fx, fy, fz, en = _make_phase_b(EP)(
        sx.reshape(r2), sy.reshape(r2), sz.reshape(r2), spf.reshape(r2),
        rx.reshape(r2), ry.reshape(r2), rz.reshape(r2),
        shifts_p[:, 0].reshape(r2), shifts_p[:, 1].reshape(r2),
        shifts_p[:, 2].reshape(r2),
        cellT, w1c, b1c, w2t, b2c, embT, woc)

    rr2 = (EP // 128, 128)
    zeros_h = jnp.zeros((NPn // 32,), f32)
    outa, outb = _make_phase_c(NPn, CR, RPW)(
        fx.reshape(rr2), fy.reshape(rr2), fz.reshape(rr2),
        snd2, rcv2, zeros_h)

    forces = (outa + outb).reshape(NPn, 4)[:N, :3]
    return en.reshape(1), forces
